# baseline (speedup 1.0000x reference)
import jax
import jax.numpy as jnp
from jax import lax
from jax.experimental import pallas as pl
from jax.experimental.pallas import tpu as pltpu

_C = 96
_W = 224
_BR = 112


def _build_placed(cm_col):
    cc = lax.broadcasted_iota(jnp.int32, (_C, _C), 0)
    kk = lax.broadcasted_iota(jnp.int32, (_C, _C), 1)
    tri_le = (kk <= cc).astype(jnp.float32)
    rank_inc = jnp.dot(tri_le, cm_col,
                       preferred_element_type=jnp.float32)
    total_kept = jnp.sum(cm_col)
    rank = rank_inc - 1.0
    kkf = kk.astype(jnp.float32)
    placed = jnp.where(rank == kkf, 1.0, 0.0) * cm_col
    pad = jnp.where((cc == 0) & (kkf >= total_kept), 1.0, 0.0)
    return placed + pad


def _mask_body(x_ref, p_ref, acc_ref):
    b = pl.program_id(0)

    @pl.when(b == 0)
    def _init():
        acc_ref[...] = jnp.zeros_like(acc_ref)

    nz = (x_ref[...] != 0.0).astype(jnp.float32)
    acc_ref[...] = jnp.maximum(acc_ref[...], jnp.max(nz, axis=(0, 1)))

    @pl.when(b == pl.num_programs(0) - 1)
    def _finalize():
        cm_col = jnp.max(acc_ref[...], axis=1, keepdims=True)
        p_ref[...] = _build_placed(cm_col)


def _gather_body(p_ref, x_ref, o_ref):
    p = p_ref[...]
    for i in range(_BR):
        o_ref[0, 0, i] = lax.dot_general(
            p, x_ref[0, 0, i],
            dimension_numbers=(((0,), (0,)), ((), ())),
            preferred_element_type=jnp.float32)


def kernel(x_inaux, x_outaux):
    xi = x_inaux.transpose(0, 1, 3, 2)
    xo = x_outaux.transpose(0, 1, 2, 4, 3)

    placed = pl.pallas_call(
        _mask_body,
        grid=(4,),
        in_specs=[pl.BlockSpec((1, 224, _C, _W), lambda b: (b, 0, 0, 0))],
        out_specs=pl.BlockSpec((_C, _C), lambda b: (0, 0)),
        out_shape=jax.ShapeDtypeStruct((_C, _C), jnp.float32),
        scratch_shapes=[pltpu.VMEM((_C, _W), jnp.float32)],
        compiler_params=pltpu.CompilerParams(
            dimension_semantics=("arbitrary",)),
    )(xi)

    out_t = pl.pallas_call(
        _gather_body,
        grid=(4, 2, 224 // _BR),
        in_specs=[
            pl.BlockSpec((_C, _C), lambda b, t, r: (0, 0)),
            pl.BlockSpec((1, 1, _BR, _C, _W),
                         lambda b, t, r: (b, t, r, 0, 0)),
        ],
        out_specs=pl.BlockSpec((1, 1, _BR, _C, _W),
                               lambda b, t, r: (b, t, r, 0, 0)),
        out_shape=jax.ShapeDtypeStruct(xo.shape, jnp.float32),
        compiler_params=pltpu.CompilerParams(
            dimension_semantics=("parallel", "parallel", "parallel")),
    )(placed, xo)

    return out_t.transpose(0, 1, 2, 4, 3)


# speedup vs baseline: 1.0064x; 1.0064x over previous
import jax
import jax.numpy as jnp
from jax import lax
from jax.experimental import pallas as pl
from jax.experimental.pallas import tpu as pltpu

_C = 96
_W = 224
_BR = 112


def _build_placed(cm_col):
    cc = lax.broadcasted_iota(jnp.int32, (_C, _C), 0)
    kk = lax.broadcasted_iota(jnp.int32, (_C, _C), 1)
    tri_le = (kk <= cc).astype(jnp.float32)
    rank_inc = jnp.dot(tri_le, cm_col,
                       preferred_element_type=jnp.float32)
    total_kept = jnp.sum(cm_col)
    rank = rank_inc - 1.0
    kkf = kk.astype(jnp.float32)
    placed = jnp.where(rank == kkf, 1.0, 0.0) * cm_col
    pad = jnp.where((cc == 0) & (kkf >= total_kept), 1.0, 0.0)
    return placed + pad


def _mask_body(x_ref, p_ref, acc_ref):
    b = pl.program_id(0)
    r = pl.program_id(1)

    @pl.when((b == 0) & (r == 0))
    def _init():
        acc_ref[...] = jnp.zeros_like(acc_ref)

    nz = (x_ref[...] != 0.0).astype(jnp.float32)
    acc_ref[...] = jnp.maximum(acc_ref[...], jnp.max(nz, axis=(0, 1)))

    @pl.when((b == pl.num_programs(0) - 1) & (r == pl.num_programs(1) - 1))
    def _finalize():
        cm_col = jnp.max(acc_ref[...], axis=1, keepdims=True)
        p_ref[...] = _build_placed(cm_col)


def _gather_body(p_ref, x_ref, o_ref):
    p = p_ref[...]
    for i in range(_BR):
        o_ref[0, 0, i] = lax.dot_general(
            p, x_ref[0, 0, i],
            dimension_numbers=(((0,), (0,)), ((), ())),
            preferred_element_type=jnp.float32)


def kernel(x_inaux, x_outaux):
    xi = x_inaux.transpose(0, 1, 3, 2)
    xo = x_outaux.transpose(0, 1, 2, 4, 3)

    placed = pl.pallas_call(
        _mask_body,
        grid=(4, 224 // _BR),
        in_specs=[pl.BlockSpec((1, _BR, _C, _W), lambda b, r: (b, r, 0, 0))],
        out_specs=pl.BlockSpec((_C, _C), lambda b, r: (0, 0)),
        out_shape=jax.ShapeDtypeStruct((_C, _C), jnp.float32),
        scratch_shapes=[pltpu.VMEM((_C, _W), jnp.float32)],
        compiler_params=pltpu.CompilerParams(
            dimension_semantics=("arbitrary", "arbitrary")),
    )(xi)

    out_t = pl.pallas_call(
        _gather_body,
        grid=(4, 2, 224 // _BR),
        in_specs=[
            pl.BlockSpec((_C, _C), lambda b, t, r: (0, 0)),
            pl.BlockSpec((1, 1, _BR, _C, _W),
                         lambda b, t, r: (b, t, r, 0, 0)),
        ],
        out_specs=pl.BlockSpec((1, 1, _BR, _C, _W),
                               lambda b, t, r: (b, t, r, 0, 0)),
        out_shape=jax.ShapeDtypeStruct(xo.shape, jnp.float32),
        compiler_params=pltpu.CompilerParams(
            dimension_semantics=("parallel", "parallel", "parallel")),
    )(placed, xo)

    return out_t.transpose(0, 1, 2, 4, 3)
